# Initial kernel scaffold; baseline (speedup 1.0000x reference)
import jax
import jax.numpy as jnp
from jax.experimental import pallas as pl


def kernel(query, table, W):
    raise NotImplementedError("write your pallas kernel here")


# speedup vs baseline: 2.7076x; 2.7076x over previous
import functools

import jax
import jax.numpy as jnp
from jax import lax
from jax.experimental import pallas as pl
from jax.experimental.pallas import tpu as pltpu
from jax.experimental.pallas import tpu_sc as plsc

B = 16384
L = 50
D = 64
LANES = 16
NC = 2
NS = 16
NW = NC * NS
BPW = B // NW
GB = 8
GPI = GB * L
NG = BPW // GB

_mesh = plsc.VectorSubcoreMesh(core_axis_name="c", subcore_axis_name="s")


@functools.partial(
    pl.kernel,
    mesh=_mesh,
    out_type=jax.ShapeDtypeStruct((B, D), jnp.float32),
    scratch_types=[
        pltpu.VMEM((BPW * L,), jnp.int32),
        pltpu.VMEM((GPI, D), jnp.float32),
        pltpu.VMEM((GPI, D), jnp.float32),
        pltpu.VMEM((GB, D), jnp.float32),
        pltpu.SemaphoreType.DMA,
        pltpu.SemaphoreType.DMA,
    ],
    compiler_params=pltpu.CompilerParams(use_tc_tiling_on_sc=False),
)
def _gather_sum(qf_hbm, table_hbm, out_hbm, idx_v, rows0, rows1, stage, sem0, sem1):
    wid = lax.axis_index("s") * NC + lax.axis_index("c")
    base = wid * BPW

    pltpu.sync_copy(qf_hbm.at[pl.ds(base * L, BPW * L)], idx_v)

    def gather(g, buf, sem):
        return pltpu.make_async_copy(
            table_hbm.at[idx_v.at[pl.ds(g * GPI, GPI)]], buf, sem)

    def compute(g, buf):
        for j in range(GB):
            def body(l, accs, j=j, buf=buf):
                r = j * L + l
                return tuple(accs[c] + buf[r, pl.ds(c * LANES, LANES)]
                             for c in range(D // LANES))
            accs = lax.fori_loop(
                0, L, body,
                tuple(jnp.zeros((LANES,), jnp.float32)
                      for _ in range(D // LANES)))
            for c in range(D // LANES):
                stage[j, pl.ds(c * LANES, LANES)] = accs[c]
        pltpu.sync_copy(stage, out_hbm.at[pl.ds(base + g * GB, GB)])

    gather(0, rows0, sem0).start()

    def body(i, carry):
        g = 2 * i
        gather(g + 1, rows1, sem1).start()
        gather(g, rows0, sem0).wait()
        compute(g, rows0)

        @pl.when(g + 2 < NG)
        def _():
            gather(g + 2, rows0, sem0).start()

        gather(g + 1, rows1, sem1).wait()
        compute(g + 1, rows1)
        return carry

    lax.fori_loop(0, NG // 2, body, 0)


def _mm_body(x_ref, w_ref, o_ref):
    o_ref[...] = lax.dot_general(
        x_ref[...], w_ref[...],
        dimension_numbers=(((1,), (1,)), ((), ())),
        preferred_element_type=jnp.float32)


def _linear(x, w):
    return pl.pallas_call(
        _mm_body,
        grid=(8,),
        in_specs=[
            pl.BlockSpec((B // 8, D), lambda i: (i, 0)),
            pl.BlockSpec((D, D), lambda i: (0, 0)),
        ],
        out_specs=pl.BlockSpec((B // 8, D), lambda i: (i, 0)),
        out_shape=jax.ShapeDtypeStruct((B, D), jnp.float32),
    )(x, w)


def kernel(query, table, W):
    qf = jnp.reshape(query.astype(jnp.int32), (B * L,))
    summed = _gather_sum(qf, table)
    return _linear(summed, W)


# speedup vs baseline: 2.7649x; 1.0212x over previous
import functools

import jax
import jax.numpy as jnp
from jax import lax
from jax.experimental import pallas as pl
from jax.experimental.pallas import tpu as pltpu
from jax.experimental.pallas import tpu_sc as plsc

B = 16384
L = 50
D = 64
LANES = 16
NC = 2
NS = 16
NW = NC * NS
BPW = B // NW
GB = 8
GPI = GB * L
NG = BPW // GB

_mesh = plsc.VectorSubcoreMesh(core_axis_name="c", subcore_axis_name="s")


@functools.partial(
    pl.kernel,
    mesh=_mesh,
    out_type=jax.ShapeDtypeStruct((B, D), jnp.float32),
    scratch_types=[
        pltpu.VMEM((BPW * L,), jnp.int32),
        pltpu.VMEM((GPI, D), jnp.float32),
        pltpu.VMEM((GPI, D), jnp.float32),
        pltpu.VMEM((GB, D), jnp.float32),
        pltpu.VMEM((GB, D), jnp.float32),
        pltpu.SemaphoreType.DMA,
        pltpu.SemaphoreType.DMA,
        pltpu.SemaphoreType.DMA,
        pltpu.SemaphoreType.DMA,
    ],
    compiler_params=pltpu.CompilerParams(use_tc_tiling_on_sc=False),
)
def _gather_sum(qf_hbm, table_hbm, out_hbm, idx_v, rows0, rows1,
                stage0, stage1, sem0, sem1, semo0, semo1):
    wid = lax.axis_index("s") * NC + lax.axis_index("c")
    base = wid * BPW
    NCOL = D // LANES

    pltpu.sync_copy(qf_hbm.at[pl.ds(base * L, BPW * L)], idx_v)

    def gather(g, buf, sem):
        return pltpu.make_async_copy(
            table_hbm.at[idx_v.at[pl.ds(g * GPI, GPI)]], buf, sem)

    def out_copy(g, stg, sem):
        return pltpu.make_async_copy(
            stg, out_hbm.at[pl.ds(base + g * GB, GB)], sem)

    def compute(g, buf, stg, semo):
        @pl.when(g >= 2)
        def _():
            out_copy(g - 2, stg, semo).wait()

        def body(l, accs):
            new = []
            for j in range(GB):
                r = j * L + l
                for c in range(NCOL):
                    new.append(accs[j * NCOL + c]
                               + buf[r, pl.ds(c * LANES, LANES)])
            return tuple(new)

        accs = lax.fori_loop(
            0, L, body,
            tuple(jnp.zeros((LANES,), jnp.float32)
                  for _ in range(GB * NCOL)))
        for j in range(GB):
            for c in range(NCOL):
                stg[j, pl.ds(c * LANES, LANES)] = accs[j * NCOL + c]
        out_copy(g, stg, semo).start()

    gather(0, rows0, sem0).start()

    def body(i, carry):
        g = 2 * i
        gather(g + 1, rows1, sem1).start()
        gather(g, rows0, sem0).wait()
        compute(g, rows0, stage0, semo0)

        @pl.when(g + 2 < NG)
        def _():
            gather(g + 2, rows0, sem0).start()

        gather(g + 1, rows1, sem1).wait()
        compute(g + 1, rows1, stage1, semo1)
        return carry

    lax.fori_loop(0, NG // 2, body, 0)

    out_copy(NG - 2, stage0, semo0).wait()
    out_copy(NG - 1, stage1, semo1).wait()


def _mm_body(x_ref, w_ref, o_ref):
    o_ref[...] = lax.dot_general(
        x_ref[...], w_ref[...],
        dimension_numbers=(((1,), (1,)), ((), ())),
        preferred_element_type=jnp.float32)


def _linear(x, w):
    return pl.pallas_call(
        _mm_body,
        grid=(8,),
        in_specs=[
            pl.BlockSpec((B // 8, D), lambda i: (i, 0)),
            pl.BlockSpec((D, D), lambda i: (0, 0)),
        ],
        out_specs=pl.BlockSpec((B // 8, D), lambda i: (i, 0)),
        out_shape=jax.ShapeDtypeStruct((B, D), jnp.float32),
    )(x, w)


def kernel(query, table, W):
    qf = jnp.reshape(query.astype(jnp.int32), (B * L,))
    summed = _gather_sum(qf, table)
    return _linear(summed, W)
